# Initial kernel scaffold; baseline (speedup 1.0000x reference)
import jax
import jax.numpy as jnp
from jax.experimental import pallas as pl


def kernel(x, pe):
    raise NotImplementedError("write your pallas kernel here")


# speedup vs baseline: 1.0687x; 1.0687x over previous
import functools

import jax
import jax.numpy as jnp
from jax import lax
from jax.experimental import pallas as pl
from jax.experimental.pallas import tpu as pltpu
from jax.experimental.pallas import tpu_sc as plsc

D_MODEL = 1024
MAX_LEN = 2048
BATCH = 4

NC = 2
NS = 16
NW = NC * NS

T_PER_W = MAX_LEN // NW
C = 16
CHUNKS = T_PER_W // C
CW = C * D_MODEL
GROUPS = CW // 16


def _pe_add_kernel(x_hbm, pe_hbm, out_hbm, pe_buf, x_buf):
    wid = lax.axis_index("s") * NC + lax.axis_index("c")
    t_base = wid * T_PER_W

    def fma_body(i, _):
        sl = pl.ds(i * 16, 16)
        x_buf[sl] = x_buf[sl] + pe_buf[sl] * 0.002
        return _

    for c in range(CHUNKS):
        t0 = t_base + c * C
        pltpu.sync_copy(pe_hbm.at[pl.ds(t0 * D_MODEL, CW)], pe_buf)
        for b in range(BATCH):
            off = (b * MAX_LEN + t0) * D_MODEL
            pltpu.sync_copy(x_hbm.at[pl.ds(off, CW)], x_buf)
            lax.fori_loop(0, GROUPS, fma_body, None)
            pltpu.sync_copy(x_buf, out_hbm.at[pl.ds(off, CW)])


@jax.jit
def _pe_add(x_flat, pe_flat):
    mesh = plsc.VectorSubcoreMesh(core_axis_name="c", subcore_axis_name="s")
    return pl.kernel(
        _pe_add_kernel,
        out_type=jax.ShapeDtypeStruct((BATCH * MAX_LEN * D_MODEL,), jnp.float32),
        mesh=mesh,
        scratch_types=[
            pltpu.VMEM((CW,), jnp.float32),
            pltpu.VMEM((CW,), jnp.float32),
        ],
    )(x_flat, pe_flat)


def kernel(x, pe):
    bz, lens, d = x.shape
    x_flat = x.reshape(-1)
    pe_flat = pe.reshape(-1)
    out = _pe_add(x_flat, pe_flat)
    return out.reshape(bz, lens, d)


# speedup vs baseline: 1.5352x; 1.4365x over previous
import jax
import jax.numpy as jnp
from jax import lax
from jax.experimental import pallas as pl
from jax.experimental.pallas import tpu as pltpu
from jax.experimental.pallas import tpu_sc as plsc

D_MODEL = 1024
MAX_LEN = 2048
BATCH = 4

NC = 2
NS = 16
NW = NC * NS

T_PER_W = MAX_LEN // NW
C = 16
CHUNKS = T_PER_W // C
CW = C * D_MODEL
GROUPS = CW // 16
NBLK = CHUNKS * BATCH


def _pe_add_kernel(x_hbm, pe_hbm, out_hbm,
                   xb0, xb1, yb0, yb1, pb0, pb1,
                   sx0, sx1, sy0, sy1, sp0, sp1):
    wid = lax.axis_index("s") * NC + lax.axis_index("c")
    t_base = wid * T_PER_W

    xb = [xb0, xb1]
    yb = [yb0, yb1]
    pb = [pb0, pb1]
    sx = [sx0, sx1]
    sy = [sy0, sy1]
    sp = [sp0, sp1]

    def x_slice(k):
        c, b = divmod(k, BATCH)
        off = (b * MAX_LEN + t_base + c * C) * D_MODEL
        return pl.ds(off, CW)

    def pe_slice(c):
        return pl.ds((t_base + c * C) * D_MODEL, CW)

    pe_wait = [None] * CHUNKS
    x_wait = [None] * NBLK
    y_wait = [None] * NBLK
    pe_wait[0] = pltpu.async_copy(pe_hbm.at[pe_slice(0)], pb[0], sp[0])
    x_wait[0] = pltpu.async_copy(x_hbm.at[x_slice(0)], xb[0], sx[0])

    def make_fma(xbuf, ybuf, pbuf):
        def fma(i, carry):
            sl = pl.ds(i * 16, 16)
            ybuf[sl] = xbuf[sl] + pbuf[sl] * 0.002
            return carry
        return fma

    for k in range(NBLK):
        c, b = divmod(k, BATCH)
        if b == 0 and c + 1 < CHUNKS:
            pe_wait[c + 1] = pltpu.async_copy(
                pe_hbm.at[pe_slice(c + 1)], pb[(c + 1) % 2], sp[(c + 1) % 2])
        if k + 1 < NBLK:
            x_wait[k + 1] = pltpu.async_copy(
                x_hbm.at[x_slice(k + 1)], xb[(k + 1) % 2], sx[(k + 1) % 2])
        if b == 0:
            pe_wait[c].wait()
        x_wait[k].wait()
        if k >= 2:
            y_wait[k - 2].wait()
        lax.fori_loop(0, GROUPS, make_fma(xb[k % 2], yb[k % 2], pb[c % 2]),
                      None)
        y_wait[k] = pltpu.async_copy(yb[k % 2], out_hbm.at[x_slice(k)],
                                     sy[k % 2])

    y_wait[NBLK - 2].wait()
    y_wait[NBLK - 1].wait()


@jax.jit
def _pe_add(x_flat, pe_flat):
    mesh = plsc.VectorSubcoreMesh(core_axis_name="c", subcore_axis_name="s")
    return pl.kernel(
        _pe_add_kernel,
        out_type=jax.ShapeDtypeStruct((BATCH * MAX_LEN * D_MODEL,), jnp.float32),
        mesh=mesh,
        scratch_types=[pltpu.VMEM((CW,), jnp.float32)] * 6
        + [pltpu.SemaphoreType.DMA] * 6,
    )(x_flat, pe_flat)


def kernel(x, pe):
    bz, lens, d = x.shape
    out = _pe_add(x.reshape(-1), pe.reshape(-1))
    return out.reshape(bz, lens, d)


# speedup vs baseline: 1.9266x; 1.2549x over previous
import jax
import jax.numpy as jnp
from jax import lax
from jax.experimental import pallas as pl
from jax.experimental.pallas import tpu as pltpu
from jax.experimental.pallas import tpu_sc as plsc

D_MODEL = 1024
MAX_LEN = 2048
BATCH = 4

NC = 2
NS = 16
NW = NC * NS

T_PER_W = MAX_LEN // NW
C = 16
CHUNKS = T_PER_W // C
CW = C * D_MODEL
GROUPS = CW // 16
NBLK = CHUNKS * BATCH


def _pe_add_kernel(x_hbm, pe_hbm, out_hbm,
                   xb0, xb1, yb0, yb1, pb0, pb1,
                   sx0, sx1, sy0, sy1, sp0, sp1):
    wid = lax.axis_index("s") * NC + lax.axis_index("c")
    t_base = wid * T_PER_W

    xb = [xb0, xb1]
    yb = [yb0, yb1]
    pb = [pb0, pb1]
    sx = [sx0, sx1]
    sy = [sy0, sy1]
    sp = [sp0, sp1]

    def x_slice(k):
        c, b = divmod(k, BATCH)
        off = (b * MAX_LEN + t_base + c * C) * D_MODEL
        return pl.ds(off, CW)

    def pe_slice(c):
        return pl.ds((t_base + c * C) * D_MODEL, CW)

    pe_wait = [None] * CHUNKS
    x_wait = [None] * NBLK
    y_wait = [None] * NBLK
    pe_wait[0] = pltpu.async_copy(pe_hbm.at[pe_slice(0)], pb[0], sp[0])
    x_wait[0] = pltpu.async_copy(x_hbm.at[x_slice(0)], xb[0], sx[0])

    def run_fma(xbuf, ybuf, pbuf):
        @plsc.parallel_loop(0, GROUPS, unroll=8)
        def _(i):
            sl = pl.ds(i * 16, 16)
            ybuf[sl] = xbuf[sl] + pbuf[sl] * 0.002

    for k in range(NBLK):
        c, b = divmod(k, BATCH)
        if b == 0 and c + 1 < CHUNKS:
            pe_wait[c + 1] = pltpu.async_copy(
                pe_hbm.at[pe_slice(c + 1)], pb[(c + 1) % 2], sp[(c + 1) % 2])
        if k + 1 < NBLK:
            x_wait[k + 1] = pltpu.async_copy(
                x_hbm.at[x_slice(k + 1)], xb[(k + 1) % 2], sx[(k + 1) % 2])
        if b == 0:
            pe_wait[c].wait()
        x_wait[k].wait()
        if k >= 2:
            y_wait[k - 2].wait()
        run_fma(xb[k % 2], yb[k % 2], pb[c % 2])
        y_wait[k] = pltpu.async_copy(yb[k % 2], out_hbm.at[x_slice(k)],
                                     sy[k % 2])

    y_wait[NBLK - 2].wait()
    y_wait[NBLK - 1].wait()


@jax.jit
def _pe_add(x_flat, pe_flat):
    mesh = plsc.VectorSubcoreMesh(core_axis_name="c", subcore_axis_name="s")
    return pl.kernel(
        _pe_add_kernel,
        out_type=jax.ShapeDtypeStruct((BATCH * MAX_LEN * D_MODEL,), jnp.float32),
        mesh=mesh,
        scratch_types=[pltpu.VMEM((CW,), jnp.float32)] * 6
        + [pltpu.SemaphoreType.DMA] * 6,
    )(x_flat, pe_flat)


def kernel(x, pe):
    bz, lens, d = x.shape
    out = _pe_add(x.reshape(-1), pe.reshape(-1))
    return out.reshape(bz, lens, d)


# speedup vs baseline: 4.2723x; 2.2176x over previous
import jax
import jax.numpy as jnp
from jax import lax
from jax.experimental import pallas as pl
from jax.experimental.pallas import tpu as pltpu
from jax.experimental.pallas import tpu_sc as plsc

D_MODEL = 1024
MAX_LEN = 2048
BATCH = 4

NC = 2
NS = 16
NW = NC * NS

T_PER_W = MAX_LEN // NW
C = 16
CHUNKS = T_PER_W // C
GROUPS = C * D_MODEL // 16
JPR = D_MODEL // 16
NBLK = CHUNKS * BATCH


def _pe_add_kernel(x_hbm, pe_hbm, out_hbm,
                   xb0, xb1, yb0, yb1, pb0, pb1,
                   sx0, sx1, sy0, sy1, sp0, sp1):
    wid = lax.axis_index("s") * NC + lax.axis_index("c")
    t_base = wid * T_PER_W

    xb = [xb0, xb1]
    yb = [yb0, yb1]
    pb = [pb0, pb1]
    sx = [sx0, sx1]
    sy = [sy0, sy1]
    sp = [sp0, sp1]

    def xsl(k):
        c, b = divmod(k, BATCH)
        return (b, pl.ds(t_base + c * C, C), slice(None))

    pe_wait = [None] * CHUNKS
    x_wait = [None] * NBLK
    y_wait = [None] * NBLK
    pe_wait[0] = pltpu.async_copy(
        pe_hbm.at[pl.ds(t_base, C), 0, :], pb[0], sp[0])
    x_wait[0] = pltpu.async_copy(x_hbm.at[xsl(0)], xb[0], sx[0])

    def run_fma(xbuf, ybuf, pbuf):
        @plsc.parallel_loop(0, GROUPS, unroll=8)
        def _(i):
            sl = (i // JPR, pl.ds((i % JPR) * 16, 16))
            ybuf[sl] = xbuf[sl] + pbuf[sl] * 0.002

    for k in range(NBLK):
        c, b = divmod(k, BATCH)
        if b == 0 and c + 1 < CHUNKS:
            pe_wait[c + 1] = pltpu.async_copy(
                pe_hbm.at[pl.ds(t_base + (c + 1) * C, C), 0, :],
                pb[(c + 1) % 2], sp[(c + 1) % 2])
        if k + 1 < NBLK:
            x_wait[k + 1] = pltpu.async_copy(
                x_hbm.at[xsl(k + 1)], xb[(k + 1) % 2], sx[(k + 1) % 2])
        if b == 0:
            pe_wait[c].wait()
        x_wait[k].wait()
        if k >= 2:
            y_wait[k - 2].wait()
        run_fma(xb[k % 2], yb[k % 2], pb[c % 2])
        y_wait[k] = pltpu.async_copy(yb[k % 2], out_hbm.at[xsl(k)], sy[k % 2])

    y_wait[NBLK - 2].wait()
    y_wait[NBLK - 1].wait()


@jax.jit
def _pe_add(x, pe):
    mesh = plsc.VectorSubcoreMesh(core_axis_name="c", subcore_axis_name="s")
    return pl.kernel(
        _pe_add_kernel,
        out_type=jax.ShapeDtypeStruct((BATCH, MAX_LEN, D_MODEL), jnp.float32),
        mesh=mesh,
        scratch_types=[pltpu.VMEM((C, D_MODEL), jnp.float32)] * 6
        + [pltpu.SemaphoreType.DMA] * 6,
        compiler_params=pltpu.CompilerParams(use_tc_tiling_on_sc=True),
    )(x, pe)


def kernel(x, pe):
    return _pe_add(x, pe)
